# baseline (speedup 1.0000x reference)
import functools

import jax
import jax.numpy as jnp
from jax import lax
from jax.experimental import pallas as pl
from jax.experimental.pallas import tpu as pltpu
from jax.experimental.pallas import tpu_sc as plsc

N_NODES = 10000
N_EDGES = 320000
D_FEAT = 128

NC = 2
NS = 16
NW = NC * NS
L = 16

K = 128
NB = N_EDGES // K
ROW_STRIDE = 624
ROW_CHUNK = 640


def _sc_segment_sum(feats, edges, zagg, zdeg):
    mesh = plsc.VectorSubcoreMesh(core_axis_name="c", subcore_axis_name="s")

    @functools.partial(
        pl.kernel,
        mesh=mesh,
        out_type=(
            jax.ShapeDtypeStruct((NC, N_NODES, D_FEAT), jnp.float32),
            jax.ShapeDtypeStruct((NW * N_NODES,), jnp.float32),
        ),
        scratch_types=[
            pltpu.VMEM((2, K), jnp.int32),
            pltpu.VMEM((K, D_FEAT), jnp.float32),
            pltpu.VMEM((2, K), jnp.int32),
            pltpu.VMEM((K, D_FEAT), jnp.float32),
            pltpu.VMEM((N_NODES,), jnp.float32),
            pltpu.VMEM_SHARED((N_NODES, D_FEAT), jnp.float32),
            pltpu.SemaphoreType.DMA,
            pltpu.SemaphoreType.DMA,
            pltpu.SemaphoreType.DMA,
            pltpu.SemaphoreType.DMA,
            pltpu.SemaphoreType.DMA,
            pltpu.SemaphoreType.DMA,
        ],
        compiler_params=pltpu.CompilerParams(needs_layout_passes=False),
    )
    def k(feats_hbm, e_hbm, zagg_hbm, zdeg_hbm,
          agg_out, deg_out, ebufa, rowsa, ebufb, rowsb,
          degl, aggsh, isema, isemb, gsema, gsemb, ssema, ssemb):
        cid = lax.axis_index("c")
        sid = lax.axis_index("s")
        wid = sid * NC + cid

        pltpu.sync_copy(zagg_hbm,
                        aggsh.at[pl.ds(sid * ROW_STRIDE, ROW_CHUNK)])
        pltpu.sync_copy(zdeg_hbm, degl)
        plsc.subcore_barrier()

        ones = jnp.ones((L,), jnp.float32)

        def start_batch(g, ebuf, rows, isem, gsem):
            di = pltpu.async_copy(e_hbm.at[g], ebuf, isem)

            def start_gather():
                di.wait()
                return pltpu.async_copy(feats_hbm.at[ebuf.at[0]], rows, gsem)

            return start_gather

        def finish_batch(gd, ebuf, rows, ssem):
            gd.wait()
            sd = pltpu.async_copy(rows, aggsh.at[ebuf.at[1]], ssem, add=True)
            for j in range(K // L):
                idx = ebuf[1, pl.ds(j * L, L)]
                plsc.addupdate_scatter(degl, [idx], ones)
            return sd

        def body(t, carry):
            sga = start_batch(wid + (2 * t) * NW, ebufa, rowsa,
                              isema, gsema)
            sgb = start_batch(wid + (2 * t + 1) * NW, ebufb, rowsb,
                              isemb, gsemb)
            gda = sga()
            gdb = sgb()
            sda = finish_batch(gda, ebufa, rowsa, ssema)
            sdb = finish_batch(gdb, ebufb, rowsb, ssemb)
            sda.wait()
            sdb.wait()
            return carry

        lax.fori_loop(0, (NB // NW) // 2, body, 0)

        @pl.when(wid < NB % NW)
        def _():
            sga = start_batch(wid + (NB // NW) * NW, ebufa, rowsa,
                              isema, gsema)
            finish_batch(sga(), ebufa, rowsa, ssema).wait()

        plsc.subcore_barrier()

        pltpu.sync_copy(aggsh.at[pl.ds(sid * ROW_STRIDE, ROW_CHUNK)],
                        agg_out.at[cid, pl.ds(sid * ROW_STRIDE, ROW_CHUNK)])
        pltpu.sync_copy(degl, deg_out.at[pl.ds(wid * N_NODES, N_NODES)])

    return k(feats, edges, zagg, zdeg)


ROW_BLK = 400


def _self_body(x_ref, wx_ref, b_ref, o_ref):
    o_ref[...] = (
        jnp.dot(x_ref[...], wx_ref[...], preferred_element_type=jnp.float32)
        + b_ref[...]
    )


def _tc_self(n_feats, wx, b2):
    grid = (N_NODES // ROW_BLK,)
    return pl.pallas_call(
        _self_body,
        grid=grid,
        in_specs=[
            pl.BlockSpec((ROW_BLK, D_FEAT), lambda i: (i, 0)),
            pl.BlockSpec((D_FEAT, D_FEAT), lambda i: (0, 0)),
            pl.BlockSpec((1, D_FEAT), lambda i: (0, 0)),
        ],
        out_specs=pl.BlockSpec((ROW_BLK, D_FEAT), lambda i: (i, 0)),
        out_shape=jax.ShapeDtypeStruct((N_NODES, D_FEAT), jnp.float32),
    )(n_feats, wx, b2)


def _finish_body(agg_ref, deg_ref, self_ref, wm_ref, o_ref):
    agg = agg_ref[0] + agg_ref[1]
    deg = jnp.sum(deg_ref[...], axis=1)
    inv = 1.0 / jnp.maximum(deg, 1.0)
    mean = agg * inv[:, None]
    o_ref[...] = (
        jnp.dot(mean, wm_ref[...], preferred_element_type=jnp.float32)
        + self_ref[...]
    )


def _tc_finish(aggp, degp, selfp, wm):
    grid = (N_NODES // ROW_BLK,)
    return pl.pallas_call(
        _finish_body,
        grid=grid,
        in_specs=[
            pl.BlockSpec((NC, ROW_BLK, D_FEAT), lambda i: (0, i, 0)),
            pl.BlockSpec((ROW_BLK, NW), lambda i: (i, 0)),
            pl.BlockSpec((ROW_BLK, D_FEAT), lambda i: (i, 0)),
            pl.BlockSpec((D_FEAT, D_FEAT), lambda i: (0, 0)),
        ],
        out_specs=pl.BlockSpec((ROW_BLK, D_FEAT), lambda i: (i, 0)),
        out_shape=jax.ShapeDtypeStruct((N_NODES, D_FEAT), jnp.float32),
    )(aggp, degp, selfp, wm)


def kernel(n_feats, edge_index, W, b):
    edges = jnp.stack([edge_index[0].reshape(NB, K),
                       edge_index[1].reshape(NB, K)], axis=1)
    zagg = jnp.zeros((ROW_CHUNK, D_FEAT), jnp.float32)
    zdeg = jnp.zeros((N_NODES,), jnp.float32)
    aggp, degp = _sc_segment_sum(n_feats, edges, zagg, zdeg)
    degp = degp.reshape(NW, N_NODES).T
    wm = W[:, :D_FEAT].T
    wx = W[:, D_FEAT:].T
    b2 = b.reshape(1, D_FEAT)
    selfp = _tc_self(n_feats, wx, b2)
    return _tc_finish(aggp, degp, selfp, wm)


# speedup vs baseline: 1.0538x; 1.0538x over previous
import functools

import jax
import jax.numpy as jnp
from jax import lax
from jax.experimental import pallas as pl
from jax.experimental.pallas import tpu as pltpu
from jax.experimental.pallas import tpu_sc as plsc

N_NODES = 10000
N_EDGES = 320000
D_FEAT = 128

NC = 2
NS = 16
NW = NC * NS
L = 16

K = 128
NB = N_EDGES // K
ROW_STRIDE = 624
ROW_CHUNK = 640


def _sc_segment_sum(feats, edges, zagg, zdeg):
    mesh = plsc.VectorSubcoreMesh(core_axis_name="c", subcore_axis_name="s")

    @functools.partial(
        pl.kernel,
        mesh=mesh,
        out_type=(
            jax.ShapeDtypeStruct((NC, N_NODES, D_FEAT), jnp.float32),
            jax.ShapeDtypeStruct((NW * N_NODES,), jnp.float32),
        ),
        scratch_types=[
            pltpu.VMEM((2, K), jnp.int32),
            pltpu.VMEM((K, D_FEAT), jnp.float32),
            pltpu.VMEM((2, K), jnp.int32),
            pltpu.VMEM((K, D_FEAT), jnp.float32),
            pltpu.VMEM((N_NODES,), jnp.float32),
            pltpu.VMEM_SHARED((N_NODES, D_FEAT), jnp.float32),
            pltpu.SemaphoreType.DMA,
            pltpu.SemaphoreType.DMA,
            pltpu.SemaphoreType.DMA,
            pltpu.SemaphoreType.DMA,
            pltpu.SemaphoreType.DMA,
            pltpu.SemaphoreType.DMA,
        ],
        compiler_params=pltpu.CompilerParams(needs_layout_passes=False),
    )
    def k(feats_hbm, e_hbm, zagg_hbm, zdeg_hbm,
          agg_out, deg_out, ebufa, rowsa, ebufb, rowsb,
          degl, aggsh, isema, isemb, gsema, gsemb, ssema, ssemb):
        cid = lax.axis_index("c")
        sid = lax.axis_index("s")
        wid = sid * NC + cid

        pltpu.sync_copy(zagg_hbm,
                        aggsh.at[pl.ds(sid * ROW_STRIDE, ROW_CHUNK)])
        pltpu.sync_copy(zdeg_hbm, degl)
        plsc.subcore_barrier()

        ones = jnp.ones((L,), jnp.float32)

        def start_batch(g, ebuf, rows, isem, gsem):
            di = pltpu.async_copy(e_hbm.at[g], ebuf, isem)

            def start_gather():
                di.wait()
                return pltpu.async_copy(feats_hbm.at[ebuf.at[0]], rows, gsem)

            return start_gather

        def finish_batch(gd, ebuf, rows, ssem):
            gd.wait()
            sd = pltpu.async_copy(rows, aggsh.at[ebuf.at[1]], ssem, add=True)
            for j in range(K // L):
                idx = ebuf[1, pl.ds(j * L, L)]
                plsc.addupdate_scatter(degl, [idx], ones)
            return sd

        def body(t, carry):
            sga = start_batch(wid + (2 * t) * NW, ebufa, rowsa,
                              isema, gsema)
            sgb = start_batch(wid + (2 * t + 1) * NW, ebufb, rowsb,
                              isemb, gsemb)
            gda = sga()
            gdb = sgb()
            sda = finish_batch(gda, ebufa, rowsa, ssema)
            sdb = finish_batch(gdb, ebufb, rowsb, ssemb)
            sda.wait()
            sdb.wait()
            return carry

        lax.fori_loop(0, (NB // NW) // 2, body, 0)

        @pl.when(wid < NB % NW)
        def _():
            sga = start_batch(wid + (NB // NW) * NW, ebufa, rowsa,
                              isema, gsema)
            finish_batch(sga(), ebufa, rowsa, ssema).wait()

        plsc.subcore_barrier()

        pltpu.sync_copy(aggsh.at[pl.ds(sid * ROW_STRIDE, ROW_CHUNK)],
                        agg_out.at[cid, pl.ds(sid * ROW_STRIDE, ROW_CHUNK)])
        pltpu.sync_copy(degl, deg_out.at[pl.ds(wid * N_NODES, N_NODES)])

    return k(feats, edges, zagg, zdeg)


ROW_BLK = 2000


def _finish_body(agg_ref, deg_ref, x_ref, wm_ref, wx_ref, b_ref, o_ref):
    agg = agg_ref[0] + agg_ref[1]
    deg = jnp.sum(deg_ref[...], axis=1)
    inv = 1.0 / jnp.maximum(deg, 1.0)
    mean = agg * inv[:, None]
    o_ref[...] = (
        jnp.dot(mean, wm_ref[...], preferred_element_type=jnp.float32)
        + jnp.dot(x_ref[...], wx_ref[...], preferred_element_type=jnp.float32)
        + b_ref[...]
    )


def _tc_finish(aggp, degp, n_feats, wm, wx, b2):
    grid = (N_NODES // ROW_BLK,)
    return pl.pallas_call(
        _finish_body,
        grid=grid,
        in_specs=[
            pl.BlockSpec((NC, ROW_BLK, D_FEAT), lambda i: (0, i, 0)),
            pl.BlockSpec((ROW_BLK, NW), lambda i: (i, 0)),
            pl.BlockSpec((ROW_BLK, D_FEAT), lambda i: (i, 0)),
            pl.BlockSpec((D_FEAT, D_FEAT), lambda i: (0, 0)),
            pl.BlockSpec((D_FEAT, D_FEAT), lambda i: (0, 0)),
            pl.BlockSpec((1, D_FEAT), lambda i: (0, 0)),
        ],
        out_specs=pl.BlockSpec((ROW_BLK, D_FEAT), lambda i: (i, 0)),
        out_shape=jax.ShapeDtypeStruct((N_NODES, D_FEAT), jnp.float32),
    )(aggp, degp, n_feats, wm, wx, b2)


def kernel(n_feats, edge_index, W, b):
    edges = jnp.stack([edge_index[0].reshape(NB, K),
                       edge_index[1].reshape(NB, K)], axis=1)
    zagg = jnp.zeros((ROW_CHUNK, D_FEAT), jnp.float32)
    zdeg = jnp.zeros((N_NODES,), jnp.float32)
    aggp, degp = _sc_segment_sum(n_feats, edges, zagg, zdeg)
    degp = degp.reshape(NW, N_NODES).T
    wm = W[:, :D_FEAT].T
    wx = W[:, D_FEAT:].T
    b2 = b.reshape(1, D_FEAT)
    return _tc_finish(aggp, degp, n_feats, wm, wx, b2)
